# baseline (speedup 1.0000x reference)
import jax
import jax.numpy as jnp
from jax import lax
from jax.experimental import pallas as pl
from jax.experimental.pallas import tpu as pltpu
from jax.experimental.pallas import tpu_sc as plsc

EMB_DIM = 64
BATCH = 16384

NUM_CORES = 2
NUM_SUBCORES = 16
NUM_WORKERS = NUM_CORES * NUM_SUBCORES

ROWS_PER_WORKER = BATCH // NUM_WORKERS
CHUNK = 128
CHUNKS = ROWS_PER_WORKER // CHUNK


def _gather_body(serv_hbm, genr_hbm, idx_hbm, out_hbm,
                 idx_v, sbuf, gbuf, gsems, wsem):
    wid = lax.axis_index("s") * NUM_CORES + lax.axis_index("c")
    ib = pl.multiple_of(wid * 2 * CHUNKS, 2 * CHUNKS)
    ob = pl.multiple_of(wid * ROWS_PER_WORKER, ROWS_PER_WORKER)
    pltpu.sync_copy(idx_hbm.at[pl.ds(ib, 2 * CHUNKS)], idx_v)
    gathers = []
    for j in range(CHUNKS):
        rows = pl.ds(j * CHUNK, CHUNK)
        gathers.append(pltpu.async_copy(
            serv_hbm.at[idx_v.at[2 * j]], sbuf.at[rows, :], gsems.at[2 * j]))
        gathers.append(pltpu.async_copy(
            genr_hbm.at[idx_v.at[2 * j + 1]], gbuf.at[rows, :],
            gsems.at[2 * j + 1]))
    for g in gathers:
        g.wait()
    pltpu.sync_copy(sbuf, out_hbm.at[pl.ds(ob, ROWS_PER_WORKER),
                                     pl.ds(0, EMB_DIM)])
    pltpu.sync_copy(gbuf, out_hbm.at[pl.ds(ob, ROWS_PER_WORKER),
                                     pl.ds(EMB_DIM, EMB_DIM)])


@jax.jit
def _gather(emb_service, emb_genre, idx):
    mesh = plsc.VectorSubcoreMesh(core_axis_name="c", subcore_axis_name="s")
    k = pl.kernel(
        _gather_body,
        out_type=jax.ShapeDtypeStruct((BATCH, 2 * EMB_DIM), jnp.float32),
        mesh=mesh,
        scratch_types=[
            pltpu.VMEM((2 * CHUNKS, CHUNK), jnp.int32),
            pltpu.VMEM((ROWS_PER_WORKER, EMB_DIM), jnp.float32),
            pltpu.VMEM((ROWS_PER_WORKER, EMB_DIM), jnp.float32),
            pltpu.SemaphoreType.DMA((2 * CHUNKS,)),
            pltpu.SemaphoreType.DMA,
        ],
        compiler_params=pltpu.CompilerParams(use_tc_tiling_on_sc=False),
    )
    return k(emb_service, emb_genre, idx)


def kernel(x2, emb_service, emb_genre):
    xi = x2.astype(jnp.int32)
    idx = xi.T.reshape(2, BATCH // CHUNK, CHUNK).transpose(1, 0, 2)
    idx = idx.reshape(2 * BATCH // CHUNK, CHUNK)
    return _gather(emb_service, emb_genre, idx)


# speedup vs baseline: 1.0042x; 1.0042x over previous
import jax
import jax.numpy as jnp
from jax import lax
from jax.experimental import pallas as pl
from jax.experimental.pallas import tpu as pltpu
from jax.experimental.pallas import tpu_sc as plsc

EMB_DIM = 64
BATCH = 16384

NUM_CORES = 2
NUM_SUBCORES = 16
NUM_WORKERS = NUM_CORES * NUM_SUBCORES

ROWS_PER_WORKER = BATCH // NUM_WORKERS
CHUNK = 128
CHUNKS = ROWS_PER_WORKER // CHUNK


def _gather_body(serv_hbm, genr_hbm, idx_hbm, out_hbm,
                 idx_v, sbuf, gbuf, gsems, wsem):
    wid = lax.axis_index("s") * NUM_CORES + lax.axis_index("c")
    ib = pl.multiple_of(wid * 2 * CHUNKS, 2 * CHUNKS)
    ob = pl.multiple_of(wid * ROWS_PER_WORKER, ROWS_PER_WORKER)
    pltpu.sync_copy(idx_hbm.at[pl.ds(ib, 2 * CHUNKS)], idx_v)
    gathers = []
    for j in range(CHUNKS):
        rows = pl.ds(j * CHUNK, CHUNK)
        gathers.append(pltpu.async_copy(
            serv_hbm.at[idx_v.at[2 * j]], sbuf.at[rows, :], gsems.at[2 * j]))
        gathers.append(pltpu.async_copy(
            genr_hbm.at[idx_v.at[2 * j + 1]], gbuf.at[rows, :],
            gsems.at[2 * j + 1]))
    for g in gathers:
        g.wait()
    pltpu.sync_copy(sbuf, out_hbm.at[pl.ds(ob, ROWS_PER_WORKER),
                                     pl.ds(0, EMB_DIM)])
    pltpu.sync_copy(gbuf, out_hbm.at[pl.ds(ob, ROWS_PER_WORKER),
                                     pl.ds(EMB_DIM, EMB_DIM)])


@jax.jit
def _gather(emb_service, emb_genre, idx):
    mesh = plsc.VectorSubcoreMesh(core_axis_name="c", subcore_axis_name="s")
    k = pl.kernel(
        _gather_body,
        out_type=jax.ShapeDtypeStruct((BATCH, 2 * EMB_DIM), jnp.float32),
        mesh=mesh,
        scratch_types=[
            pltpu.VMEM((2 * CHUNKS, CHUNK), jnp.int32),
            pltpu.VMEM((ROWS_PER_WORKER, EMB_DIM), jnp.float32),
            pltpu.VMEM((ROWS_PER_WORKER, EMB_DIM), jnp.float32),
            pltpu.SemaphoreType.DMA((2 * CHUNKS,)),
            pltpu.SemaphoreType.DMA,
        ],
        compiler_params=pltpu.CompilerParams(use_tc_tiling_on_sc=False,
                                             skip_device_barrier=True),
    )
    return k(emb_service, emb_genre, idx)


def kernel(x2, emb_service, emb_genre):
    xi = x2.astype(jnp.int32)
    idx = xi.T.reshape(2, BATCH // CHUNK, CHUNK).transpose(1, 0, 2)
    idx = idx.reshape(2 * BATCH // CHUNK, CHUNK)
    return _gather(emb_service, emb_genre, idx)
